# baseline (speedup 1.0000x reference)
import functools

import jax
import jax.numpy as jnp
from jax import lax
from jax.experimental import pallas as pl
from jax.experimental.pallas import tpu as pltpu
from jax.experimental.pallas import tpu_sc as plsc

B = 4096
L = 200
D = 512
NCLS = 1000
VPAD = 1024

NW = 32
ROWS_PER_W = B // NW
CH = 64
NCHUNK = ROWS_PER_W // CH
NVEC = L // 16


def _sc_histogram(x):
    mesh = plsc.VectorSubcoreMesh(core_axis_name="c", subcore_axis_name="s")

    @functools.partial(
        pl.kernel,
        mesh=mesh,
        out_type=jax.ShapeDtypeStruct((B * VPAD,), jnp.float32),
        scratch_types=[
            pltpu.VMEM((CH * L,), jnp.int32),
            pltpu.VMEM((CH * VPAD,), jnp.float32),
        ],
        compiler_params=pltpu.CompilerParams(needs_layout_passes=False),
    )
    def hist_kernel(x_hbm, counts_hbm, idx_v, hist_v):
        wid = lax.axis_index("c") * 16 + lax.axis_index("s")
        ones = jnp.ones((16,), jnp.float32)
        zeros = jnp.zeros((16,), jnp.float32)
        lane = lax.iota(jnp.int32, 16)
        tail_mask = lane >= 8

        for c in range(NCHUNK):
            base = wid * ROWS_PER_W + c * CH
            pltpu.sync_copy(x_hbm.at[pl.ds(base * L, CH * L)], idx_v)

            ZUNROLL = 32

            def zero_blk(i, carry):
                for u in range(ZUNROLL):
                    hist_v[pl.ds((i * ZUNROLL + u) * 16, 16)] = zeros
                return carry

            lax.fori_loop(0, CH * VPAD // (16 * ZUNROLL), zero_blk, 0)

            def do_row(r, carry):
                rbase = r * VPAD
                for j in range(NVEC):
                    ids = idx_v[pl.ds(r * L + j * 16, 16)]
                    plsc.addupdate_scatter(hist_v, [ids + rbase], ones)
                ids = idx_v[pl.ds(r * L + L - 16, 16)]
                plsc.addupdate_scatter(hist_v, [ids + rbase], ones, mask=tail_mask)
                return carry

            lax.fori_loop(0, CH, do_row, 0)

            pltpu.sync_copy(hist_v, counts_hbm.at[pl.ds(base * VPAD, CH * VPAD)])

    return hist_kernel(x.reshape(B * L)).reshape(B, VPAD)


BB = 256


def _mlp_body(counts_ref, emb_ref, w1_ref, b1_ref, w2_ref, b2_ref, out_ref):
    c = counts_ref[...]
    denom = 200.0 - c[:, 0:1]
    s = jnp.dot(
        c.astype(jnp.bfloat16),
        emb_ref[...].astype(jnp.bfloat16),
        preferred_element_type=jnp.float32,
    )
    pooled = s / denom
    h = jnp.dot(
        pooled.astype(jnp.bfloat16),
        w1_ref[...].astype(jnp.bfloat16),
        preferred_element_type=jnp.float32,
    ) + b1_ref[...]
    h = jnp.maximum(h, 0.0)
    out_ref[...] = jnp.dot(
        h.astype(jnp.bfloat16),
        w2_ref[...].astype(jnp.bfloat16),
        preferred_element_type=jnp.float32,
    ) + b2_ref[...]


def _tc_mlp(counts, emb_z, W1, b1, W2, b2):
    return pl.pallas_call(
        _mlp_body,
        grid=(B // BB,),
        in_specs=[
            pl.BlockSpec((BB, VPAD), lambda i: (i, 0)),
            pl.BlockSpec((VPAD, D), lambda i: (0, 0)),
            pl.BlockSpec((D, D), lambda i: (0, 0)),
            pl.BlockSpec((1, D), lambda i: (0, 0)),
            pl.BlockSpec((D, NCLS), lambda i: (0, 0)),
            pl.BlockSpec((1, NCLS), lambda i: (0, 0)),
        ],
        out_specs=pl.BlockSpec((BB, NCLS), lambda i: (i, 0)),
        out_shape=jax.ShapeDtypeStruct((B, NCLS), jnp.float32),
    )(counts, emb_z, W1, b1.reshape(1, D), W2, b2.reshape(1, NCLS))


def kernel(x, emb, W1, b1, W2, b2):
    counts = _sc_histogram(x.astype(jnp.int32))
    emb_z = jnp.zeros((VPAD, D), emb.dtype).at[1:NCLS].set(emb[1:])
    return _tc_mlp(counts, emb_z, W1, b1, W2, b2)


# speedup vs baseline: 1.2106x; 1.2106x over previous
import functools

import jax
import jax.numpy as jnp
from jax import lax
from jax.experimental import pallas as pl
from jax.experimental.pallas import tpu as pltpu
from jax.experimental.pallas import tpu_sc as plsc

B = 4096
L = 200
D = 512
NCLS = 1000
VPAD = 1024

NW = 32
ROWS_PER_W = B // NW
CH = 64
NCHUNK = ROWS_PER_W // CH
NVEC = L // 16


def _sc_histogram(x):
    mesh = plsc.VectorSubcoreMesh(core_axis_name="c", subcore_axis_name="s")

    @functools.partial(
        pl.kernel,
        mesh=mesh,
        out_type=jax.ShapeDtypeStruct((B, VPAD), jnp.float32),
        scratch_types=[
            pltpu.VMEM((CH, L), jnp.int32),
            pltpu.VMEM((CH, VPAD), jnp.float32),
        ],
        compiler_params=pltpu.CompilerParams(needs_layout_passes=False),
    )
    def hist_kernel(x_hbm, counts_hbm, idx_v, hist_v):
        wid = lax.axis_index("c") * 16 + lax.axis_index("s")
        ones = jnp.ones((16,), jnp.float32)
        zeros = jnp.zeros((16,), jnp.float32)
        lane = lax.iota(jnp.int32, 16)
        tail_mask = lane >= 8

        for c in range(NCHUNK):
            base = wid * ROWS_PER_W + c * CH
            pltpu.sync_copy(x_hbm.at[pl.ds(base, CH)], idx_v)

            def zero_row(r, carry):
                for k in range(VPAD // 16):
                    hist_v[r, pl.ds(k * 16, 16)] = zeros
                return carry

            lax.fori_loop(0, CH, zero_row, 0)

            def do_row(r, carry):
                rvec = jnp.full((16,), r, jnp.int32)
                for j in range(NVEC):
                    ids = idx_v[r, pl.ds(j * 16, 16)]
                    plsc.addupdate_scatter(hist_v, [rvec, ids], ones)
                ids = idx_v[r, pl.ds(L - 16, 16)]
                plsc.addupdate_scatter(hist_v, [rvec, ids], ones, mask=tail_mask)
                return carry

            lax.fori_loop(0, CH, do_row, 0)

            pltpu.sync_copy(hist_v, counts_hbm.at[pl.ds(base, CH)])

    return hist_kernel(x)


BB = 256


def _mlp_body(counts_ref, emb_ref, w1_ref, b1_ref, w2_ref, b2_ref, out_ref):
    c = counts_ref[...]
    denom = 200.0 - c[:, 0:1]
    cv = c[:, :NCLS]
    col = lax.broadcasted_iota(jnp.int32, (BB, NCLS), 1)
    cv = jnp.where(col == 0, 0.0, cv)
    s = jnp.dot(
        cv.astype(jnp.bfloat16),
        emb_ref[...].astype(jnp.bfloat16),
        preferred_element_type=jnp.float32,
    )
    pooled = s / denom
    h = jnp.dot(
        pooled.astype(jnp.bfloat16),
        w1_ref[...].astype(jnp.bfloat16),
        preferred_element_type=jnp.float32,
    ) + b1_ref[...]
    h = jnp.maximum(h, 0.0)
    out_ref[...] = jnp.dot(
        h.astype(jnp.bfloat16),
        w2_ref[...].astype(jnp.bfloat16),
        preferred_element_type=jnp.float32,
    ) + b2_ref[...]


def _tc_mlp(counts, emb, W1, b1, W2, b2):
    return pl.pallas_call(
        _mlp_body,
        grid=(B // BB,),
        in_specs=[
            pl.BlockSpec((BB, VPAD), lambda i: (i, 0)),
            pl.BlockSpec((NCLS, D), lambda i: (0, 0)),
            pl.BlockSpec((D, D), lambda i: (0, 0)),
            pl.BlockSpec((1, D), lambda i: (0, 0)),
            pl.BlockSpec((D, NCLS), lambda i: (0, 0)),
            pl.BlockSpec((1, NCLS), lambda i: (0, 0)),
        ],
        out_specs=pl.BlockSpec((BB, NCLS), lambda i: (i, 0)),
        out_shape=jax.ShapeDtypeStruct((B, NCLS), jnp.float32),
    )(counts, emb, W1, b1.reshape(1, D), W2, b2.reshape(1, NCLS))


def kernel(x, emb, W1, b1, W2, b2):
    counts = _sc_histogram(x.astype(jnp.int32))
    return _tc_mlp(counts, emb, W1, b1, W2, b2)


# speedup vs baseline: 1.5684x; 1.2955x over previous
import functools

import jax
import jax.numpy as jnp
from jax import lax
from jax.experimental import pallas as pl
from jax.experimental.pallas import tpu as pltpu
from jax.experimental.pallas import tpu_sc as plsc

B = 4096
L = 200
D = 512
NCLS = 1000
VPAD = 1024

NW = 32
ROWS_PER_W = B // NW
CH = 64
NCHUNK = ROWS_PER_W // CH
NGRP = CH // 16


def _sc_histogram(xt):
    mesh = plsc.VectorSubcoreMesh(core_axis_name="c", subcore_axis_name="s")

    @functools.partial(
        pl.kernel,
        mesh=mesh,
        out_type=jax.ShapeDtypeStruct((B, VPAD), jnp.float32),
        scratch_types=[
            pltpu.VMEM((L, ROWS_PER_W), jnp.int32),
            pltpu.VMEM((CH, VPAD), jnp.float32),
        ],
        compiler_params=pltpu.CompilerParams(needs_layout_passes=False),
    )
    def hist_kernel(xt_hbm, counts_hbm, idx_v, hist_v):
        wid = lax.axis_index("c") * 16 + lax.axis_index("s")
        ones = jnp.ones((16,), jnp.float32)
        zeros = jnp.zeros((16,), jnp.float32)
        lane = lax.iota(jnp.int32, 16)

        pltpu.sync_copy(xt_hbm.at[:, pl.ds(wid * ROWS_PER_W, ROWS_PER_W)], idx_v)

        for c in range(NCHUNK):
            base = wid * ROWS_PER_W + c * CH

            def zero_row(r, carry):
                for k in range(VPAD // 16):
                    hist_v[r, pl.ds(k * 16, 16)] = zeros
                return carry

            lax.fori_loop(0, CH, zero_row, 0)

            def do_tok(j, carry):
                for g in range(NGRP):
                    rvec = lane + (g * 16)
                    ids = idx_v[j, pl.ds(c * CH + g * 16, 16)]
                    plsc.addupdate_scatter(hist_v, [rvec, ids], ones)
                return carry

            lax.fori_loop(0, L, do_tok, 0)

            pltpu.sync_copy(hist_v, counts_hbm.at[pl.ds(base, CH)])

    return hist_kernel(xt)


BB = 256


def _mlp_body(counts_ref, emb_ref, w1_ref, b1_ref, w2t_ref, b2_ref, outt_ref):
    c = counts_ref[...]
    denom = 200.0 - c[:, 0:1]
    cv = c[:, :NCLS]
    col = lax.broadcasted_iota(jnp.int32, (BB, NCLS), 1)
    cv = jnp.where(col == 0, 0.0, cv)
    s = jnp.dot(
        cv.astype(jnp.bfloat16),
        emb_ref[...].astype(jnp.bfloat16),
        preferred_element_type=jnp.float32,
    )
    pooled = s / denom
    h = jnp.dot(
        pooled.astype(jnp.bfloat16),
        w1_ref[...].astype(jnp.bfloat16),
        preferred_element_type=jnp.float32,
    ) + b1_ref[...]
    h = jnp.maximum(h, 0.0)
    outt_ref[...] = lax.dot_general(
        w2t_ref[...].astype(jnp.bfloat16),
        h.astype(jnp.bfloat16),
        (((1,), (1,)), ((), ())),
        preferred_element_type=jnp.float32,
    ) + b2_ref[...]


def _tc_mlp(counts, emb, W1, b1, W2t, b2):
    outt = pl.pallas_call(
        _mlp_body,
        grid=(B // BB,),
        in_specs=[
            pl.BlockSpec((BB, VPAD), lambda i: (i, 0)),
            pl.BlockSpec((NCLS, D), lambda i: (0, 0)),
            pl.BlockSpec((D, D), lambda i: (0, 0)),
            pl.BlockSpec((1, D), lambda i: (0, 0)),
            pl.BlockSpec((NCLS, D), lambda i: (0, 0)),
            pl.BlockSpec((NCLS, 1), lambda i: (0, 0)),
        ],
        out_specs=pl.BlockSpec((NCLS, BB), lambda i: (0, i)),
        out_shape=jax.ShapeDtypeStruct((NCLS, B), jnp.float32),
    )(counts, emb, W1, b1.reshape(1, D), W2t, b2.reshape(NCLS, 1))
    return outt.T


def kernel(x, emb, W1, b1, W2, b2):
    xt = jnp.swapaxes(x.astype(jnp.int32), 0, 1)
    counts = _sc_histogram(xt)
    return _tc_mlp(counts, emb, W1, b1, jnp.swapaxes(W2, 0, 1), b2)


# speedup vs baseline: 1.6524x; 1.0536x over previous
import functools

import jax
import jax.numpy as jnp
from jax import lax
from jax.experimental import pallas as pl
from jax.experimental.pallas import tpu as pltpu
from jax.experimental.pallas import tpu_sc as plsc

B = 4096
L = 200
D = 512
NCLS = 1000
VPAD = 1024

NW = 32
NHALF = 2
BH = B // NHALF
CH = BH // NW


def _sc_histogram_half(xt, half):
    mesh = plsc.VectorSubcoreMesh(core_axis_name="c", subcore_axis_name="s")

    @functools.partial(
        pl.kernel,
        mesh=mesh,
        out_type=jax.ShapeDtypeStruct((BH, VPAD), jnp.float32),
        scratch_types=[
            pltpu.VMEM((L, 2 * CH), jnp.int32),
            pltpu.VMEM((CH, VPAD), jnp.float32),
        ],
        compiler_params=pltpu.CompilerParams(needs_layout_passes=False),
    )
    def hist_kernel(xt_hbm, counts_hbm, idx_v, hist_v):
        wid = lax.axis_index("c") * 16 + lax.axis_index("s")
        ones = jnp.ones((16,), jnp.float32)
        zeros = jnp.zeros((16,), jnp.float32)
        lane = lax.iota(jnp.int32, 16)

        aligned = half * BH + (wid // 2) * (2 * CH)
        off = (wid % 2) * CH
        pltpu.sync_copy(xt_hbm.at[:, pl.ds(aligned, 2 * CH)], idx_v)

        def zero_row(r, carry):
            for k in range(VPAD // 16):
                hist_v[r, pl.ds(k * 16, 16)] = zeros
            return carry

        lax.fori_loop(0, CH, zero_row, 0)

        def do_tok(j, carry):
            for g in range(CH // 16):
                rvec = lane + (g * 16)
                ids = idx_v[j, pl.ds(off + g * 16, 16)]
                plsc.addupdate_scatter(hist_v, [rvec, ids], ones)
            return carry

        lax.fori_loop(0, L, do_tok, 0)

        pltpu.sync_copy(hist_v, counts_hbm.at[pl.ds(wid * CH, CH)])

    return hist_kernel(xt)


BB = 256
GH = BH // BB


def _mlp_body(_, counts_ref, emb_ref, w1_ref, b1_ref, w2t_ref, b2_ref, outt_ref):
    c = counts_ref[...]
    denom = 200.0 - c[:, 0:1]
    cv = c[:, :NCLS]
    col = lax.broadcasted_iota(jnp.int32, (BB, NCLS), 1)
    cv = jnp.where(col == 0, 0.0, cv)
    s = jnp.dot(
        cv.astype(jnp.bfloat16),
        emb_ref[...].astype(jnp.bfloat16),
        preferred_element_type=jnp.float32,
    )
    pooled = s / denom
    h = jnp.dot(
        pooled.astype(jnp.bfloat16),
        w1_ref[...].astype(jnp.bfloat16),
        preferred_element_type=jnp.float32,
    ) + b1_ref[...]
    h = jnp.maximum(h, 0.0)
    outt_ref[...] = lax.dot_general(
        w2t_ref[...].astype(jnp.bfloat16),
        h.astype(jnp.bfloat16),
        (((1,), (1,)), ((), ())),
        preferred_element_type=jnp.float32,
    ) + b2_ref[...]


def _tc_mlp_half(prev_outt, counts_h, emb, W1, b1r, W2t, b2r, half):
    body = _mlp_body if prev_outt is not None else (
        lambda c, e, w1, bb1, w2t, bb2, o: _mlp_body(None, c, e, w1, bb1, w2t, bb2, o)
    )
    specs = [
        pl.BlockSpec((BB, VPAD), lambda i: (i, 0)),
        pl.BlockSpec((NCLS, D), lambda i: (0, 0)),
        pl.BlockSpec((D, D), lambda i: (0, 0)),
        pl.BlockSpec((1, D), lambda i: (0, 0)),
        pl.BlockSpec((NCLS, D), lambda i: (0, 0)),
        pl.BlockSpec((NCLS, 1), lambda i: (0, 0)),
    ]
    args = (counts_h, emb, W1, b1r, W2t, b2r)
    aliases = {}
    if prev_outt is not None:
        specs = [pl.BlockSpec(memory_space=pl.MemorySpace.ANY)] + specs
        args = (prev_outt,) + args
        aliases = {0: 0}
    return pl.pallas_call(
        body,
        grid=(GH,),
        in_specs=specs,
        out_specs=pl.BlockSpec((NCLS, BB), lambda i, h=half: (0, h * GH + i)),
        out_shape=jax.ShapeDtypeStruct((NCLS, B), jnp.float32),
        input_output_aliases=aliases,
    )(*args)


def kernel(x, emb, W1, b1, W2, b2):
    xt = jnp.swapaxes(x.astype(jnp.int32), 0, 1)
    W2t = jnp.swapaxes(W2, 0, 1)
    b1r = b1.reshape(1, D)
    b2r = b2.reshape(NCLS, 1)

    counts0 = _sc_histogram_half(xt, 0)
    counts1 = _sc_histogram_half(xt, 1)
    outt = _tc_mlp_half(None, counts0, emb, W1, b1r, W2t, b2r, 0)
    outt = _tc_mlp_half(outt, counts1, emb, W1, b1r, W2t, b2r, 1)
    return outt.T
